# baseline (speedup 1.0000x reference)
import jax
import jax.numpy as jnp
from jax.experimental import pallas as pl

B, N, S, K = 16, 2048, 512, 32
DP = 64
CIN = 3 + DP
EPS = 1e-5
SB = 512
RB = 2048
NPOS = B * S * K
NBLK = NPOS // RB


def _fps_kernel(x3_ref, f0_ref, nx3_ref):
    x = x3_ref[...]
    lane_n = jax.lax.broadcasted_iota(jnp.int32, (B, N), 1)
    lane_s = jax.lax.broadcasted_iota(jnp.int32, (3, B, S), 2)
    far0 = f0_ref[...]
    dist0 = jnp.full((B, N), 1e10, dtype=jnp.float32)
    acc0 = jnp.zeros((3, B, S), dtype=jnp.float32)

    def body(i, carry):
        dist, far, acc = carry
        mask = lane_n == far
        centroid = jnp.sum(jnp.where(mask[None], x, 0.0), axis=2,
                           keepdims=True)
        acc = jnp.where(lane_s == i, centroid, acc)
        diff = x - centroid
        d = jnp.sum(diff * diff, axis=0)
        dist = jnp.minimum(dist, d)
        far_new = jnp.argmax(dist, axis=1, keepdims=True).astype(jnp.int32)
        return dist, far_new, acc

    _, _, acc = jax.lax.fori_loop(0, S, body, (dist0, far0, acc0))
    nx3_ref[...] = acc


def _feat_mm_kernel(x_ref, w_ref, f1_ref):
    f1_ref[...] = jax.lax.dot_general(x_ref[...], w_ref[...],
                                      (((1,), (0,)), ((), ())),
                                      preferred_element_type=jnp.float32)


def _knn_kernel(xT_ref, nxr_ref, f1_ref, wx_ref, b1_ref, z_ref, s_ref, q_ref):
    crow = nxr_ref[0]
    d = jnp.zeros((SB, N), jnp.float32)
    for ch in range(3):
        diff = crow[:, ch:ch + 1] - xT_ref[0, ch:ch + 1, :]
        d = d + diff * diff
    f1b = f1_ref[0]
    offset = b1_ref[...] - jax.lax.dot_general(
        crow, wx_ref[...], (((1,), (0,)), ((), ())),
        preferred_element_type=jnp.float32)
    iota_n = jax.lax.broadcasted_iota(jnp.int32, (SB, N), 1)
    ps = jnp.zeros((1, 64), jnp.float32)
    pq = jnp.zeros((1, 64), jnp.float32)
    for k in range(K):
        sel = jnp.argmin(d, axis=1, keepdims=True).astype(jnp.int32)
        mask = iota_n == sel
        row = jax.lax.dot_general(mask.astype(jnp.float32), f1b,
                                  (((1,), (0,)), ((), ())),
                                  preferred_element_type=jnp.float32) + offset
        z_ref[0, k, :, :] = row
        ps = ps + jnp.sum(row, axis=0, keepdims=True)
        pq = pq + jnp.sum(row * row, axis=0, keepdims=True)
        d = jnp.where(mask, jnp.inf, d)

    @pl.when((pl.program_id(0) == 0) & (pl.program_id(1) == 0))
    def _():
        s_ref[...] = ps
        q_ref[...] = pq

    @pl.when((pl.program_id(0) > 0) | (pl.program_id(1) > 0))
    def _():
        s_ref[...] += ps
        q_ref[...] += pq


def _bn_mm_stats_kernel(z_ref, s_ref, q_ref, g_ref, be_ref, w_ref, b_ref,
                        z2_ref, s2_ref, q2_ref):
    mean = s_ref[...] / NPOS
    var = q_ref[...] / NPOS - mean * mean
    scale = g_ref[...] / jnp.sqrt(var + EPS)
    shift = be_ref[...] - mean * scale
    x = jnp.maximum(z_ref[...] * scale + shift, 0.0)
    z2 = jax.lax.dot_general(x, w_ref[...], (((1,), (0,)), ((), ())),
                             preferred_element_type=jnp.float32) + b_ref[...]
    z2_ref[...] = z2
    ps = jnp.sum(z2, axis=0, keepdims=True)
    pq = jnp.sum(z2 * z2, axis=0, keepdims=True)

    @pl.when(pl.program_id(0) == 0)
    def _():
        s2_ref[...] = ps
        q2_ref[...] = pq

    @pl.when(pl.program_id(0) > 0)
    def _():
        s2_ref[...] += ps
        q2_ref[...] += pq


def _bn_pool_kernel(z_ref, s_ref, q_ref, g_ref, be_ref, out_ref):
    mean = s_ref[...] / NPOS
    var = q_ref[...] / NPOS - mean * mean
    scale = g_ref[...] / jnp.sqrt(var + EPS)
    shift = be_ref[...] - mean * scale
    x = jnp.maximum(z_ref[0] * scale + shift, 0.0)
    out_ref[0] = jnp.max(x, axis=0)


def kernel(xyz, points, farthest0, W1, b1, g1, beta1, W2, b2, g2, beta2,
           W3, b3, g3, beta3):
    xT = jnp.transpose(xyz, (0, 2, 1))
    x3 = jnp.transpose(xyz, (2, 0, 1))
    f0 = farthest0.astype(jnp.int32).reshape(B, 1)

    nx3 = pl.pallas_call(
        _fps_kernel,
        out_shape=jax.ShapeDtypeStruct((3, B, S), jnp.float32),
    )(x3, f0)
    new_xyz = jnp.transpose(nx3, (1, 2, 0))

    featf = jnp.concatenate([xyz, points], axis=2).reshape(B * N, CIN)
    F1 = pl.pallas_call(
        _feat_mm_kernel,
        out_shape=jax.ShapeDtypeStruct((B * N, 64), jnp.float32),
    )(featf, W1.T)
    F1r = F1.reshape(B, N, 64)

    Z1, s1, q1 = pl.pallas_call(
        _knn_kernel,
        grid=(B, S // SB),
        in_specs=[
            pl.BlockSpec((1, 3, N), lambda b, s: (b, 0, 0)),
            pl.BlockSpec((1, SB, 3), lambda b, s: (b, s, 0)),
            pl.BlockSpec((1, N, 64), lambda b, s: (b, 0, 0)),
            pl.BlockSpec((3, 64), lambda b, s: (0, 0)),
            pl.BlockSpec((1, 64), lambda b, s: (0, 0)),
        ],
        out_specs=[
            pl.BlockSpec((1, K, SB, 64), lambda b, s: (b, 0, s, 0)),
            pl.BlockSpec((1, 64), lambda b, s: (0, 0)),
            pl.BlockSpec((1, 64), lambda b, s: (0, 0)),
        ],
        out_shape=[
            jax.ShapeDtypeStruct((B, K, S, 64), jnp.float32),
            jax.ShapeDtypeStruct((1, 64), jnp.float32),
            jax.ShapeDtypeStruct((1, 64), jnp.float32),
        ],
    )(xT, new_xyz, F1r, W1[:, :3].T, b1.reshape(1, -1))

    Z1f = Z1.reshape(NPOS, 64)

    def stats_specs(cout):
        return ([pl.BlockSpec((RB, cout), lambda i: (i, 0)),
                 pl.BlockSpec((1, cout), lambda i: (0, 0)),
                 pl.BlockSpec((1, cout), lambda i: (0, 0))],
                [jax.ShapeDtypeStruct((NPOS, cout), jnp.float32),
                 jax.ShapeDtypeStruct((1, cout), jnp.float32),
                 jax.ShapeDtypeStruct((1, cout), jnp.float32)])

    def bn_layer(Z, s, q, g, be, wT, b, cin, cout):
        o_specs, o_shapes = stats_specs(cout)
        return pl.pallas_call(
            _bn_mm_stats_kernel,
            grid=(NBLK,),
            in_specs=[
                pl.BlockSpec((RB, cin), lambda i: (i, 0)),
                pl.BlockSpec((1, cin), lambda i: (0, 0)),
                pl.BlockSpec((1, cin), lambda i: (0, 0)),
                pl.BlockSpec((1, cin), lambda i: (0, 0)),
                pl.BlockSpec((1, cin), lambda i: (0, 0)),
                pl.BlockSpec((cin, cout), lambda i: (0, 0)),
                pl.BlockSpec((1, cout), lambda i: (0, 0)),
            ],
            out_specs=o_specs,
            out_shape=o_shapes,
        )(Z, s, q, g.reshape(1, -1), be.reshape(1, -1), wT, b.reshape(1, -1))

    Z2, s2, q2 = bn_layer(Z1f, s1, q1, g1, beta1, W2.T, b2, 64, 64)
    Z3, s3, q3 = bn_layer(Z2, s2, q2, g2, beta2, W3.T, b3, 64, 128)

    Z3r = Z3.reshape(B, K, S, 128)
    pooled = pl.pallas_call(
        _bn_pool_kernel,
        grid=(B,),
        in_specs=[
            pl.BlockSpec((1, K, S, 128), lambda b: (b, 0, 0, 0)),
            pl.BlockSpec((1, 128), lambda b: (0, 0)),
            pl.BlockSpec((1, 128), lambda b: (0, 0)),
            pl.BlockSpec((1, 128), lambda b: (0, 0)),
            pl.BlockSpec((1, 128), lambda b: (0, 0)),
        ],
        out_specs=pl.BlockSpec((1, S, 128), lambda b: (b, 0, 0)),
        out_shape=jax.ShapeDtypeStruct((B, S, 128), jnp.float32),
    )(Z3r, s3, q3, g3.reshape(1, -1), beta3.reshape(1, -1))

    new_points = jnp.transpose(pooled, (0, 2, 1))
    return (new_xyz, new_points)


# speedup vs baseline: 1.2285x; 1.2285x over previous
import jax
import jax.numpy as jnp
from jax.experimental import pallas as pl

B, N, S, K = 16, 2048, 512, 32
DP = 64
CIN = 3 + DP
EPS = 1e-5
SB = 256
RB = 2048
NPOS = B * S * K
NBLK = NPOS // RB


def _fps_kernel(x3_ref, f0_ref, nx3_ref):
    x = x3_ref[...]
    lane_n = jax.lax.broadcasted_iota(jnp.int32, (B, N), 1)
    lane_s = jax.lax.broadcasted_iota(jnp.int32, (3, B, S), 2)
    far0 = f0_ref[...]
    dist0 = jnp.full((B, N), 1e10, dtype=jnp.float32)
    acc0 = jnp.zeros((3, B, S), dtype=jnp.float32)

    def body(i, carry):
        dist, far, acc = carry
        mask = lane_n == far
        centroid = jnp.sum(jnp.where(mask[None], x, 0.0), axis=2,
                           keepdims=True)
        acc = jnp.where(lane_s == i, centroid, acc)
        diff = x - centroid
        d = jnp.sum(diff * diff, axis=0)
        dist = jnp.minimum(dist, d)
        far_new = jnp.argmax(dist, axis=1, keepdims=True).astype(jnp.int32)
        return dist, far_new, acc

    _, _, acc = jax.lax.fori_loop(0, S, body, (dist0, far0, acc0))
    nx3_ref[...] = acc


def _feat_mm_kernel(x_ref, w_ref, f1_ref):
    f1_ref[...] = jax.lax.dot_general(x_ref[...], w_ref[...],
                                      (((1,), (0,)), ((), ())),
                                      preferred_element_type=jnp.float32)


def _knn_kernel(xT_ref, nxr_ref, f1_ref, wx_ref, b1_ref, z_ref, s_ref, q_ref):
    crow = nxr_ref[0]
    d = jnp.zeros((SB, N), jnp.float32)
    for ch in range(3):
        diff = crow[:, ch:ch + 1] - xT_ref[0, ch:ch + 1, :]
        d = d + diff * diff
    f1b = f1_ref[0]
    offset = b1_ref[...] - jax.lax.dot_general(
        crow, wx_ref[...], (((1,), (0,)), ((), ())),
        preferred_element_type=jnp.float32)
    iota_n = jax.lax.broadcasted_iota(jnp.int32, (SB, N), 1)
    ps = jnp.zeros((1, 64), jnp.float32)
    pq = jnp.zeros((1, 64), jnp.float32)
    for k in range(K):
        sel = jnp.argmin(d, axis=1, keepdims=True).astype(jnp.int32)
        mask = iota_n == sel
        row = jax.lax.dot_general(mask.astype(jnp.float32), f1b,
                                  (((1,), (0,)), ((), ())),
                                  preferred_element_type=jnp.float32) + offset
        z_ref[0, k, :, :] = row
        ps = ps + jnp.sum(row, axis=0, keepdims=True)
        pq = pq + jnp.sum(row * row, axis=0, keepdims=True)
        d = jnp.where(mask, jnp.inf, d)

    @pl.when((pl.program_id(0) == 0) & (pl.program_id(1) == 0))
    def _():
        s_ref[...] = ps
        q_ref[...] = pq

    @pl.when((pl.program_id(0) > 0) | (pl.program_id(1) > 0))
    def _():
        s_ref[...] += ps
        q_ref[...] += pq


def _bn_mm_stats_kernel(z_ref, s_ref, q_ref, g_ref, be_ref, w_ref, b_ref,
                        z2_ref, s2_ref, q2_ref):
    mean = s_ref[...] / NPOS
    var = q_ref[...] / NPOS - mean * mean
    scale = g_ref[...] / jnp.sqrt(var + EPS)
    shift = be_ref[...] - mean * scale
    x = jnp.maximum(z_ref[...] * scale + shift, 0.0)
    z2 = jax.lax.dot_general(x, w_ref[...], (((1,), (0,)), ((), ())),
                             preferred_element_type=jnp.float32) + b_ref[...]
    z2_ref[...] = z2
    ps = jnp.sum(z2, axis=0, keepdims=True)
    pq = jnp.sum(z2 * z2, axis=0, keepdims=True)

    @pl.when(pl.program_id(0) == 0)
    def _():
        s2_ref[...] = ps
        q2_ref[...] = pq

    @pl.when(pl.program_id(0) > 0)
    def _():
        s2_ref[...] += ps
        q2_ref[...] += pq


def _bn_mm_minmax_kernel(z_ref, s_ref, q_ref, g_ref, be_ref, w_ref, b_ref,
                         mx_ref, mn_ref, s2_ref, q2_ref):
    mean = s_ref[...] / NPOS
    var = q_ref[...] / NPOS - mean * mean
    scale = g_ref[...] / jnp.sqrt(var + EPS)
    shift = be_ref[...] - mean * scale
    x = jnp.maximum(z_ref[...] * scale + shift, 0.0)
    z2 = jax.lax.dot_general(x, w_ref[...], (((1,), (0,)), ((), ())),
                             preferred_element_type=jnp.float32) + b_ref[...]
    ps = jnp.sum(z2, axis=0, keepdims=True)
    pq = jnp.sum(z2 * z2, axis=0, keepdims=True)
    zb = z2.reshape(RB // S, S, -1)
    pmax = jnp.max(zb, axis=0)
    pmin = jnp.min(zb, axis=0)

    @pl.when(pl.program_id(1) == 0)
    def _():
        mx_ref[...] = pmax
        mn_ref[...] = pmin

    @pl.when(pl.program_id(1) > 0)
    def _():
        mx_ref[...] = jnp.maximum(mx_ref[...], pmax)
        mn_ref[...] = jnp.minimum(mn_ref[...], pmin)

    first = (pl.program_id(0) == 0) & (pl.program_id(1) == 0)

    @pl.when(first)
    def _():
        s2_ref[...] = ps
        q2_ref[...] = pq

    @pl.when(jnp.logical_not(first))
    def _():
        s2_ref[...] += ps
        q2_ref[...] += pq


def _affine_relu_max_kernel(mx_ref, mn_ref, s_ref, q_ref, g_ref, be_ref,
                            out_ref):
    mean = s_ref[...] / NPOS
    var = q_ref[...] / NPOS - mean * mean
    scale = g_ref[...] / jnp.sqrt(var + EPS)
    shift = be_ref[...] - mean * scale
    a = mx_ref[...] * scale + shift
    b = mn_ref[...] * scale + shift
    out_ref[...] = jnp.maximum(jnp.maximum(a, b), 0.0)


def kernel(xyz, points, farthest0, W1, b1, g1, beta1, W2, b2, g2, beta2,
           W3, b3, g3, beta3):
    xT = jnp.transpose(xyz, (0, 2, 1))
    x3 = jnp.transpose(xyz, (2, 0, 1))
    f0 = farthest0.astype(jnp.int32).reshape(B, 1)

    nx3 = pl.pallas_call(
        _fps_kernel,
        out_shape=jax.ShapeDtypeStruct((3, B, S), jnp.float32),
    )(x3, f0)
    new_xyz = jnp.transpose(nx3, (1, 2, 0))

    featf = jnp.concatenate([xyz, points], axis=2).reshape(B * N, CIN)
    F1 = pl.pallas_call(
        _feat_mm_kernel,
        out_shape=jax.ShapeDtypeStruct((B * N, 64), jnp.float32),
    )(featf, W1.T)
    F1r = F1.reshape(B, N, 64)

    Z1, s1, q1 = pl.pallas_call(
        _knn_kernel,
        grid=(B, S // SB),
        in_specs=[
            pl.BlockSpec((1, 3, N), lambda b, s: (b, 0, 0)),
            pl.BlockSpec((1, SB, 3), lambda b, s: (b, s, 0)),
            pl.BlockSpec((1, N, 64), lambda b, s: (b, 0, 0)),
            pl.BlockSpec((3, 64), lambda b, s: (0, 0)),
            pl.BlockSpec((1, 64), lambda b, s: (0, 0)),
        ],
        out_specs=[
            pl.BlockSpec((1, K, SB, 64), lambda b, s: (b, 0, s, 0)),
            pl.BlockSpec((1, 64), lambda b, s: (0, 0)),
            pl.BlockSpec((1, 64), lambda b, s: (0, 0)),
        ],
        out_shape=[
            jax.ShapeDtypeStruct((B, K, S, 64), jnp.float32),
            jax.ShapeDtypeStruct((1, 64), jnp.float32),
            jax.ShapeDtypeStruct((1, 64), jnp.float32),
        ],
    )(xT, new_xyz, F1r, W1[:, :3].T, b1.reshape(1, -1))

    Z1f = Z1.reshape(NPOS, 64)

    def stats_specs(cout):
        return ([pl.BlockSpec((RB, cout), lambda i: (i, 0)),
                 pl.BlockSpec((1, cout), lambda i: (0, 0)),
                 pl.BlockSpec((1, cout), lambda i: (0, 0))],
                [jax.ShapeDtypeStruct((NPOS, cout), jnp.float32),
                 jax.ShapeDtypeStruct((1, cout), jnp.float32),
                 jax.ShapeDtypeStruct((1, cout), jnp.float32)])

    def bn_layer(Z, s, q, g, be, wT, b, cin, cout):
        o_specs, o_shapes = stats_specs(cout)
        return pl.pallas_call(
            _bn_mm_stats_kernel,
            grid=(NBLK,),
            in_specs=[
                pl.BlockSpec((RB, cin), lambda i: (i, 0)),
                pl.BlockSpec((1, cin), lambda i: (0, 0)),
                pl.BlockSpec((1, cin), lambda i: (0, 0)),
                pl.BlockSpec((1, cin), lambda i: (0, 0)),
                pl.BlockSpec((1, cin), lambda i: (0, 0)),
                pl.BlockSpec((cin, cout), lambda i: (0, 0)),
                pl.BlockSpec((1, cout), lambda i: (0, 0)),
            ],
            out_specs=o_specs,
            out_shape=o_shapes,
        )(Z, s, q, g.reshape(1, -1), be.reshape(1, -1), wT, b.reshape(1, -1))

    Z2, s2, q2 = bn_layer(Z1f, s1, q1, g1, beta1, W2.T, b2, 64, 64)

    JB = (K * S) // RB
    zmx, zmn, s3, q3 = pl.pallas_call(
        _bn_mm_minmax_kernel,
        grid=(B, JB),
        in_specs=[
            pl.BlockSpec((RB, 64), lambda b, j: (b * JB + j, 0)),
            pl.BlockSpec((1, 64), lambda b, j: (0, 0)),
            pl.BlockSpec((1, 64), lambda b, j: (0, 0)),
            pl.BlockSpec((1, 64), lambda b, j: (0, 0)),
            pl.BlockSpec((1, 64), lambda b, j: (0, 0)),
            pl.BlockSpec((64, 128), lambda b, j: (0, 0)),
            pl.BlockSpec((1, 128), lambda b, j: (0, 0)),
        ],
        out_specs=[
            pl.BlockSpec((S, 128), lambda b, j: (b, 0)),
            pl.BlockSpec((S, 128), lambda b, j: (b, 0)),
            pl.BlockSpec((1, 128), lambda b, j: (0, 0)),
            pl.BlockSpec((1, 128), lambda b, j: (0, 0)),
        ],
        out_shape=[
            jax.ShapeDtypeStruct((B * S, 128), jnp.float32),
            jax.ShapeDtypeStruct((B * S, 128), jnp.float32),
            jax.ShapeDtypeStruct((1, 128), jnp.float32),
            jax.ShapeDtypeStruct((1, 128), jnp.float32),
        ],
    )(Z2, s2, q2, g2.reshape(1, -1), beta2.reshape(1, -1), W3.T,
      b3.reshape(1, -1))

    PB = (B * S) // 8
    pooled = pl.pallas_call(
        _affine_relu_max_kernel,
        grid=(8,),
        in_specs=[
            pl.BlockSpec((PB, 128), lambda i: (i, 0)),
            pl.BlockSpec((PB, 128), lambda i: (i, 0)),
            pl.BlockSpec((1, 128), lambda i: (0, 0)),
            pl.BlockSpec((1, 128), lambda i: (0, 0)),
            pl.BlockSpec((1, 128), lambda i: (0, 0)),
            pl.BlockSpec((1, 128), lambda i: (0, 0)),
        ],
        out_specs=pl.BlockSpec((PB, 128), lambda i: (i, 0)),
        out_shape=jax.ShapeDtypeStruct((B * S, 128), jnp.float32),
    )(zmx, zmn, s3, q3, g3.reshape(1, -1), beta3.reshape(1, -1))

    new_points = jnp.transpose(pooled.reshape(B, S, 128), (0, 2, 1))
    return (new_xyz, new_points)


# speedup vs baseline: 1.3128x; 1.0686x over previous
import jax
import jax.numpy as jnp
from jax.experimental import pallas as pl

B, N, S, K = 16, 2048, 512, 32
DP = 64
CIN = 3 + DP
EPS = 1e-5
SB = 256
RB = 4096
NPOS = B * S * K
NBLK = NPOS // RB


def _fps_kernel(x3_ref, f0_ref, feat_ref, w_ref, nx3_ref, f1_ref):
    f1_ref[...] = jax.lax.dot_general(feat_ref[...], w_ref[...],
                                      (((1,), (0,)), ((), ())),
                                      preferred_element_type=jnp.float32)
    x = x3_ref[...]
    lane_n = jax.lax.broadcasted_iota(jnp.int32, (B, N), 1)
    lane_s = jax.lax.broadcasted_iota(jnp.int32, (3, B, S), 2)
    far0 = f0_ref[...]
    dist0 = jnp.full((B, N), 1e10, dtype=jnp.float32)
    acc0 = jnp.zeros((3, B, S), dtype=jnp.float32)

    def body(i, carry):
        dist, far, acc = carry
        mask = lane_n == far
        centroid = jnp.sum(jnp.where(mask[None], x, 0.0), axis=2,
                           keepdims=True)
        acc = jnp.where(lane_s == i, centroid, acc)
        diff = x - centroid
        d = jnp.sum(diff * diff, axis=0)
        dist = jnp.minimum(dist, d)
        far_new = jnp.argmax(dist, axis=1, keepdims=True).astype(jnp.int32)
        return dist, far_new, acc

    _, _, acc = jax.lax.fori_loop(0, S, body, (dist0, far0, acc0))
    nx3_ref[...] = acc


def _knn_kernel(xT_ref, nxr_ref, f1_ref, wx_ref, b1_ref, z_ref, s_ref, q_ref):
    crow = nxr_ref[0]
    d = jnp.zeros((SB, N), jnp.float32)
    for ch in range(3):
        diff = crow[:, ch:ch + 1] - xT_ref[0, ch:ch + 1, :]
        d = d + diff * diff
    f1b = f1_ref[0]
    offset = b1_ref[...] - jax.lax.dot_general(
        crow, wx_ref[...], (((1,), (0,)), ((), ())),
        preferred_element_type=jnp.float32)
    iota_n = jax.lax.broadcasted_iota(jnp.int32, (SB, N), 1)
    ps = jnp.zeros((1, 64), jnp.float32)
    pq = jnp.zeros((1, 64), jnp.float32)
    for k in range(K):
        sel = jnp.argmin(d, axis=1, keepdims=True).astype(jnp.int32)
        mask = iota_n == sel
        row = jax.lax.dot_general(mask.astype(jnp.float32), f1b,
                                  (((1,), (0,)), ((), ())),
                                  preferred_element_type=jnp.float32) + offset
        z_ref[0, k, :, :] = row
        ps = ps + jnp.sum(row, axis=0, keepdims=True)
        pq = pq + jnp.sum(row * row, axis=0, keepdims=True)
        d = jnp.where(mask, jnp.inf, d)

    @pl.when((pl.program_id(0) == 0) & (pl.program_id(1) == 0))
    def _():
        s_ref[...] = ps
        q_ref[...] = pq

    @pl.when((pl.program_id(0) > 0) | (pl.program_id(1) > 0))
    def _():
        s_ref[...] += ps
        q_ref[...] += pq


def _bn_mm_stats_kernel(z_ref, s_ref, q_ref, g_ref, be_ref, w_ref, b_ref,
                        z2_ref, s2_ref, q2_ref):
    mean = s_ref[...] / NPOS
    var = q_ref[...] / NPOS - mean * mean
    scale = g_ref[...] / jnp.sqrt(var + EPS)
    shift = be_ref[...] - mean * scale
    x = jnp.maximum(z_ref[...] * scale + shift, 0.0)
    z2 = jax.lax.dot_general(x, w_ref[...], (((1,), (0,)), ((), ())),
                             preferred_element_type=jnp.float32) + b_ref[...]
    z2_ref[...] = z2
    ps = jnp.sum(z2, axis=0, keepdims=True)
    pq = jnp.sum(z2 * z2, axis=0, keepdims=True)

    @pl.when(pl.program_id(0) == 0)
    def _():
        s2_ref[...] = ps
        q2_ref[...] = pq

    @pl.when(pl.program_id(0) > 0)
    def _():
        s2_ref[...] += ps
        q2_ref[...] += pq


def _bn_mm_minmax_kernel(z_ref, s_ref, q_ref, g_ref, be_ref, w_ref, b_ref,
                         mx_ref, mn_ref, s2_ref, q2_ref):
    mean = s_ref[...] / NPOS
    var = q_ref[...] / NPOS - mean * mean
    scale = g_ref[...] / jnp.sqrt(var + EPS)
    shift = be_ref[...] - mean * scale
    x = jnp.maximum(z_ref[...] * scale + shift, 0.0)
    z2 = jax.lax.dot_general(x, w_ref[...], (((1,), (0,)), ((), ())),
                             preferred_element_type=jnp.float32) + b_ref[...]
    ps = jnp.sum(z2, axis=0, keepdims=True)
    pq = jnp.sum(z2 * z2, axis=0, keepdims=True)
    zb = z2.reshape(RB // S, S, -1)
    pmax = jnp.max(zb, axis=0)
    pmin = jnp.min(zb, axis=0)

    @pl.when(pl.program_id(1) == 0)
    def _():
        mx_ref[...] = pmax
        mn_ref[...] = pmin

    @pl.when(pl.program_id(1) > 0)
    def _():
        mx_ref[...] = jnp.maximum(mx_ref[...], pmax)
        mn_ref[...] = jnp.minimum(mn_ref[...], pmin)

    first = (pl.program_id(0) == 0) & (pl.program_id(1) == 0)

    @pl.when(first)
    def _():
        s2_ref[...] = ps
        q2_ref[...] = pq

    @pl.when(jnp.logical_not(first))
    def _():
        s2_ref[...] += ps
        q2_ref[...] += pq


def _affine_relu_max_kernel(mx_ref, mn_ref, s_ref, q_ref, g_ref, be_ref,
                            out_ref):
    mean = s_ref[...] / NPOS
    var = q_ref[...] / NPOS - mean * mean
    scale = g_ref[...] / jnp.sqrt(var + EPS)
    shift = be_ref[...] - mean * scale
    a = mx_ref[...] * scale + shift
    b = mn_ref[...] * scale + shift
    out_ref[...] = jnp.maximum(jnp.maximum(a, b), 0.0)


def kernel(xyz, points, farthest0, W1, b1, g1, beta1, W2, b2, g2, beta2,
           W3, b3, g3, beta3):
    xT = jnp.transpose(xyz, (0, 2, 1))
    x3 = jnp.transpose(xyz, (2, 0, 1))
    f0 = farthest0.astype(jnp.int32).reshape(B, 1)

    featf = jnp.concatenate([xyz, points], axis=2).reshape(B * N, CIN)
    nx3, F1 = pl.pallas_call(
        _fps_kernel,
        out_shape=[jax.ShapeDtypeStruct((3, B, S), jnp.float32),
                   jax.ShapeDtypeStruct((B * N, 64), jnp.float32)],
    )(x3, f0, featf, W1.T)
    new_xyz = jnp.transpose(nx3, (1, 2, 0))
    F1r = F1.reshape(B, N, 64)

    Z1, s1, q1 = pl.pallas_call(
        _knn_kernel,
        grid=(B, S // SB),
        in_specs=[
            pl.BlockSpec((1, 3, N), lambda b, s: (b, 0, 0)),
            pl.BlockSpec((1, SB, 3), lambda b, s: (b, s, 0)),
            pl.BlockSpec((1, N, 64), lambda b, s: (b, 0, 0)),
            pl.BlockSpec((3, 64), lambda b, s: (0, 0)),
            pl.BlockSpec((1, 64), lambda b, s: (0, 0)),
        ],
        out_specs=[
            pl.BlockSpec((1, K, SB, 64), lambda b, s: (b, 0, s, 0)),
            pl.BlockSpec((1, 64), lambda b, s: (0, 0)),
            pl.BlockSpec((1, 64), lambda b, s: (0, 0)),
        ],
        out_shape=[
            jax.ShapeDtypeStruct((B, K, S, 64), jnp.float32),
            jax.ShapeDtypeStruct((1, 64), jnp.float32),
            jax.ShapeDtypeStruct((1, 64), jnp.float32),
        ],
    )(xT, new_xyz, F1r, W1[:, :3].T, b1.reshape(1, -1))

    Z1f = Z1.reshape(NPOS, 64)

    def stats_specs(cout):
        return ([pl.BlockSpec((RB, cout), lambda i: (i, 0)),
                 pl.BlockSpec((1, cout), lambda i: (0, 0)),
                 pl.BlockSpec((1, cout), lambda i: (0, 0))],
                [jax.ShapeDtypeStruct((NPOS, cout), jnp.float32),
                 jax.ShapeDtypeStruct((1, cout), jnp.float32),
                 jax.ShapeDtypeStruct((1, cout), jnp.float32)])

    def bn_layer(Z, s, q, g, be, wT, b, cin, cout):
        o_specs, o_shapes = stats_specs(cout)
        return pl.pallas_call(
            _bn_mm_stats_kernel,
            grid=(NBLK,),
            in_specs=[
                pl.BlockSpec((RB, cin), lambda i: (i, 0)),
                pl.BlockSpec((1, cin), lambda i: (0, 0)),
                pl.BlockSpec((1, cin), lambda i: (0, 0)),
                pl.BlockSpec((1, cin), lambda i: (0, 0)),
                pl.BlockSpec((1, cin), lambda i: (0, 0)),
                pl.BlockSpec((cin, cout), lambda i: (0, 0)),
                pl.BlockSpec((1, cout), lambda i: (0, 0)),
            ],
            out_specs=o_specs,
            out_shape=o_shapes,
        )(Z, s, q, g.reshape(1, -1), be.reshape(1, -1), wT, b.reshape(1, -1))

    Z2, s2, q2 = bn_layer(Z1f, s1, q1, g1, beta1, W2.T, b2, 64, 64)

    JB = (K * S) // RB
    zmx, zmn, s3, q3 = pl.pallas_call(
        _bn_mm_minmax_kernel,
        grid=(B, JB),
        in_specs=[
            pl.BlockSpec((RB, 64), lambda b, j: (b * JB + j, 0)),
            pl.BlockSpec((1, 64), lambda b, j: (0, 0)),
            pl.BlockSpec((1, 64), lambda b, j: (0, 0)),
            pl.BlockSpec((1, 64), lambda b, j: (0, 0)),
            pl.BlockSpec((1, 64), lambda b, j: (0, 0)),
            pl.BlockSpec((64, 128), lambda b, j: (0, 0)),
            pl.BlockSpec((1, 128), lambda b, j: (0, 0)),
        ],
        out_specs=[
            pl.BlockSpec((S, 128), lambda b, j: (b, 0)),
            pl.BlockSpec((S, 128), lambda b, j: (b, 0)),
            pl.BlockSpec((1, 128), lambda b, j: (0, 0)),
            pl.BlockSpec((1, 128), lambda b, j: (0, 0)),
        ],
        out_shape=[
            jax.ShapeDtypeStruct((B * S, 128), jnp.float32),
            jax.ShapeDtypeStruct((B * S, 128), jnp.float32),
            jax.ShapeDtypeStruct((1, 128), jnp.float32),
            jax.ShapeDtypeStruct((1, 128), jnp.float32),
        ],
    )(Z2, s2, q2, g2.reshape(1, -1), beta2.reshape(1, -1), W3.T,
      b3.reshape(1, -1))

    PB = (B * S) // 8
    pooled = pl.pallas_call(
        _affine_relu_max_kernel,
        grid=(8,),
        in_specs=[
            pl.BlockSpec((PB, 128), lambda i: (i, 0)),
            pl.BlockSpec((PB, 128), lambda i: (i, 0)),
            pl.BlockSpec((1, 128), lambda i: (0, 0)),
            pl.BlockSpec((1, 128), lambda i: (0, 0)),
            pl.BlockSpec((1, 128), lambda i: (0, 0)),
            pl.BlockSpec((1, 128), lambda i: (0, 0)),
        ],
        out_specs=pl.BlockSpec((PB, 128), lambda i: (i, 0)),
        out_shape=jax.ShapeDtypeStruct((B * S, 128), jnp.float32),
    )(zmx, zmn, s3, q3, g3.reshape(1, -1), beta3.reshape(1, -1))

    new_points = jnp.transpose(pooled.reshape(B, S, 128), (0, 2, 1))
    return (new_xyz, new_points)
